# baseline (speedup 1.0000x reference)
import jax
import jax.numpy as jnp
from jax.experimental import pallas as pl

_B, _S, _D, _E, _K = 4, 2048, 4096, 64, 8
_ALPHA = 0.1
_RATIO = 0.9
_DEAD = 0.01
_TOK_BLOCK = 512


def _router_body(h_ref, wct_ref, weights_ref, experts_ref):
    h = h_ref[...]
    wct = wct_ref[...]
    logits = jax.lax.dot_general(
        wct, h, (((1,), (1,)), ((), ())), preferred_element_type=jnp.float32
    )
    raw = logits[:_E, :]
    wm = logits[_E:, :]

    m = jnp.max(raw, axis=0, keepdims=True)
    e = jnp.exp(raw - m)
    probs = e / jnp.sum(e, axis=0, keepdims=True)

    iota = jax.lax.broadcasted_iota(jnp.int32, probs.shape, 0)

    p1 = jnp.max(probs, axis=0, keepdims=True)
    idx1 = jnp.min(jnp.where(probs == p1, iota, _E), axis=0, keepdims=True)
    p2 = jnp.max(jnp.where(iota == idx1, -jnp.inf, probs), axis=0, keepdims=True)
    safe = (p2 / (p1 + 1e-9)) >= _RATIO

    mean_raw = jnp.mean(raw, axis=0, keepdims=True)
    var_raw = jnp.sum((raw - mean_raw) ** 2, axis=0, keepdims=True) / (_E - 1)
    mean_wm = jnp.mean(wm, axis=0, keepdims=True)
    var_wm = jnp.sum((wm - mean_wm) ** 2, axis=0, keepdims=True) / (_E - 1)
    scale = jnp.sqrt(var_raw) / (jnp.sqrt(var_wm) + 1e-9)

    combined = safe & (jnp.abs(wm) >= _DEAD)
    scores = jnp.where(combined, raw + _ALPHA * wm * scale, raw)

    idx_rows = []
    val_rows = []
    for _ in range(_K):
        mk = jnp.max(scores, axis=0, keepdims=True)
        idx = jnp.min(jnp.where(scores == mk, iota, _E), axis=0, keepdims=True)
        onehot = iota == idx
        val = jnp.sum(jnp.where(onehot, raw, 0.0), axis=0, keepdims=True)
        idx_rows.append(idx)
        val_rows.append(val)
        scores = jnp.where(onehot, -jnp.inf, scores)
    sel = jnp.concatenate(idx_rows, axis=0)
    vals = jnp.concatenate(val_rows, axis=0)

    mv = jnp.max(vals, axis=0, keepdims=True)
    ev = jnp.exp(vals - mv)
    weights_ref[...] = ev / jnp.sum(ev, axis=0, keepdims=True)
    experts_ref[...] = sel


@jax.jit
def kernel(hidden_states, W_gate, secret_projection):
    n_tok = _B * _S
    h2 = hidden_states.reshape(n_tok, _D)
    wct = jnp.concatenate([W_gate, secret_projection.T], axis=0)

    grid = (n_tok // _TOK_BLOCK,)
    weights_t, experts_t = pl.pallas_call(
        _router_body,
        grid=grid,
        in_specs=[
            pl.BlockSpec((_TOK_BLOCK, _D), lambda i: (i, 0)),
            pl.BlockSpec((2 * _E, _D), lambda i: (0, 0)),
        ],
        out_specs=[
            pl.BlockSpec((_K, _TOK_BLOCK), lambda i: (0, i)),
            pl.BlockSpec((_K, _TOK_BLOCK), lambda i: (0, i)),
        ],
        out_shape=[
            jax.ShapeDtypeStruct((_K, n_tok), jnp.float32),
            jax.ShapeDtypeStruct((_K, n_tok), jnp.int32),
        ],
    )(h2, wct)
    return (
        weights_t.T.reshape(_B, _S, _K),
        experts_t.T.reshape(_B, _S, _K),
    )


# speedup vs baseline: 1.0690x; 1.0690x over previous
import jax
import jax.numpy as jnp
from jax.experimental import pallas as pl

_B, _S, _D, _E, _K = 4, 2048, 4096, 64, 8
_ALPHA = 0.1
_RATIO = 0.9
_DEAD = 0.01
_TOK_BLOCK = 1024


def _router_body(h_ref, wct_ref, weights_ref, experts_ref):
    h = h_ref[...]
    wct = wct_ref[...]
    logits = jax.lax.dot_general(
        wct, h, (((1,), (1,)), ((), ())), preferred_element_type=jnp.float32
    )
    raw = logits[:_E, :]
    wm = logits[_E:, :]

    m = jnp.max(raw, axis=0, keepdims=True)
    e = jnp.exp(raw - m)
    probs = e / jnp.sum(e, axis=0, keepdims=True)

    iota = jax.lax.broadcasted_iota(jnp.int32, probs.shape, 0)

    p1 = jnp.max(probs, axis=0, keepdims=True)
    idx1 = jnp.min(jnp.where(probs == p1, iota, _E), axis=0, keepdims=True)
    p2 = jnp.max(jnp.where(iota == idx1, -jnp.inf, probs), axis=0, keepdims=True)
    safe = (p2 / (p1 + 1e-9)) >= _RATIO

    mean_raw = jnp.mean(raw, axis=0, keepdims=True)
    var_raw = jnp.sum((raw - mean_raw) ** 2, axis=0, keepdims=True) / (_E - 1)
    mean_wm = jnp.mean(wm, axis=0, keepdims=True)
    var_wm = jnp.sum((wm - mean_wm) ** 2, axis=0, keepdims=True) / (_E - 1)
    scale = jnp.sqrt(var_raw) / (jnp.sqrt(var_wm) + 1e-9)

    combined = safe & (jnp.abs(wm) >= _DEAD)
    scores = jnp.where(combined, raw + _ALPHA * wm * scale, raw)

    idx_rows = []
    val_rows = []
    for _ in range(_K):
        mk = jnp.max(scores, axis=0, keepdims=True)
        idx = jnp.min(jnp.where(scores == mk, iota, _E), axis=0, keepdims=True)
        onehot = iota == idx
        val = jnp.sum(jnp.where(onehot, raw, 0.0), axis=0, keepdims=True)
        idx_rows.append(idx)
        val_rows.append(val)
        scores = jnp.where(onehot, -jnp.inf, scores)
    sel = jnp.concatenate(idx_rows, axis=0)
    vals = jnp.concatenate(val_rows, axis=0)

    mv = jnp.max(vals, axis=0, keepdims=True)
    ev = jnp.exp(vals - mv)
    weights_ref[...] = ev / jnp.sum(ev, axis=0, keepdims=True)
    experts_ref[...] = sel


@jax.jit
def kernel(hidden_states, W_gate, secret_projection):
    n_tok = _B * _S
    h2 = hidden_states.reshape(n_tok, _D)
    wct = jnp.concatenate([W_gate, secret_projection.T], axis=0)

    grid = (n_tok // _TOK_BLOCK,)
    weights_t, experts_t = pl.pallas_call(
        _router_body,
        grid=grid,
        in_specs=[
            pl.BlockSpec((_TOK_BLOCK, _D), lambda i: (i, 0)),
            pl.BlockSpec((2 * _E, _D), lambda i: (0, 0)),
        ],
        out_specs=[
            pl.BlockSpec((_K, _TOK_BLOCK), lambda i: (0, i)),
            pl.BlockSpec((_K, _TOK_BLOCK), lambda i: (0, i)),
        ],
        out_shape=[
            jax.ShapeDtypeStruct((_K, n_tok), jnp.float32),
            jax.ShapeDtypeStruct((_K, n_tok), jnp.int32),
        ],
    )(h2, wct)
    return (
        weights_t.T.reshape(_B, _S, _K),
        experts_t.T.reshape(_B, _S, _K),
    )
